# baseline (speedup 1.0000x reference)
import functools

import jax
import jax.numpy as jnp
from jax import lax
from jax.experimental import pallas as pl
from jax.experimental.pallas import tpu as pltpu
from jax.experimental.pallas import tpu_sc as plsc

_B = 16
_SEG = 2048
_D = 1024
_IMAX = 64
_HALF = _D // 2
_LANES = 16
_GROUPS = _HALF // _LANES

_RSC = 64
_CH = 64
_NCH = _RSC // _CH

_TC_BLOCKS = ((64, 1), (128, 1), (256, 1), (512, 1), (1024, 1))


def _sc_prefix_sums(hidden_states, instruction_lens):
    mesh = plsc.VectorSubcoreMesh(core_axis_name="c", subcore_axis_name="s")

    @functools.partial(
        pl.kernel,
        out_type=jax.ShapeDtypeStruct((_B, _D), jnp.float32),
        mesh=mesh,
        compiler_params=pltpu.CompilerParams(needs_layout_passes=False),
        scratch_types=[
            pltpu.VMEM((_CH, _HALF), jnp.float32),
            pltpu.VMEM((_HALF,), jnp.float32),
            pltpu.VMEM((_B,), jnp.int32),
            pltpu.SemaphoreType.DMA,
        ],
    )
    def k(hs_hbm, ilen_hbm, out_hbm, buf, acc, ilen_vmem, sem0):
        wid = lax.axis_index("s") * 2 + lax.axis_index("c")
        seg = wid // 2
        col0 = (wid % 2) * _HALF
        row0 = seg * _SEG
        cp = pltpu.async_copy(
            hs_hbm.at[pl.ds(row0, _CH), pl.ds(col0, _HALF)], buf, sem0
        )
        pltpu.sync_copy(ilen_hbm, ilen_vmem)
        nvec = plsc.load_gather(ilen_vmem, [jnp.full((_LANES,), seg, jnp.int32)])
        zero = jnp.zeros((_LANES,), jnp.float32)
        cp.wait()

        @pl.loop(0, _GROUPS)
        def _(g):
            sl = pl.ds(g * _LANES, _LANES)
            p = [zero] * 8
            for r in range(_CH):
                x = jnp.where(
                    lax.broadcast(r, (_LANES,)) >= nvec, buf[r, sl], zero
                )
                p[r % 8] = p[r % 8] + x
            acc[sl] = ((p[0] + p[1]) + (p[2] + p[3])) + (
                (p[4] + p[5]) + (p[6] + p[7])
            )

        pltpu.sync_copy(acc, out_hbm.at[seg, pl.ds(col0, _HALF)])

    return k(hidden_states, instruction_lens)


def _tc_dense_body(*refs):
    hs_refs, out_ref = refs[:-1], refs[-1]
    acc = jnp.sum(hs_refs[0][...], axis=0, keepdims=True)
    for j in range(1, len(hs_refs)):
        acc = acc + jnp.sum(hs_refs[j][...], axis=0, keepdims=True)
    out_ref[0] = acc


def _tc_combine_body(plen_ref, ilen_ref, dense_ref, sc_ref, out_ref):
    for b in range(_B):
        denom = (plen_ref[b] - ilen_ref[b]).astype(jnp.float32)
        mean = (dense_ref[b, 0] + sc_ref[b]) / denom
        norm = jnp.maximum(jnp.sqrt(jnp.sum(mean * mean)), 1e-12)
        out_ref[b] = mean / norm


def kernel(hidden_states, prompt_lens, instruction_lens):
    sc_sums = _sc_prefix_sums(hidden_states, instruction_lens)
    dense_sums = pl.pallas_call(
        _tc_dense_body,
        grid=(_B,),
        in_specs=[
            pl.BlockSpec(
                (rows, _D),
                functools.partial(
                    lambda o, b: (b * (_SEG // o[0]) + o[1], 0), (rows, off)
                ),
            )
            for rows, off in _TC_BLOCKS
        ],
        out_specs=pl.BlockSpec((1, 1, _D), lambda b: (b, 0, 0)),
        out_shape=jax.ShapeDtypeStruct((_B, 1, _D), jnp.float32),
        compiler_params=pltpu.CompilerParams(dimension_semantics=("parallel",)),
    )(*([hidden_states] * len(_TC_BLOCKS)))
    return pl.pallas_call(
        _tc_combine_body,
        in_specs=[
            pl.BlockSpec(memory_space=pltpu.SMEM),
            pl.BlockSpec(memory_space=pltpu.SMEM),
            pl.BlockSpec((_B, 1, _D), lambda: (0, 0, 0)),
            pl.BlockSpec((_B, _D), lambda: (0, 0)),
        ],
        out_specs=pl.BlockSpec((_B, _D), lambda: (0, 0)),
        out_shape=jax.ShapeDtypeStruct((_B, _D), jnp.float32),
    )(prompt_lens, instruction_lens, dense_sums, sc_sums)


# speedup vs baseline: 1.0132x; 1.0132x over previous
import functools

import jax
import jax.numpy as jnp
from jax import lax
from jax.experimental import pallas as pl
from jax.experimental.pallas import tpu as pltpu
from jax.experimental.pallas import tpu_sc as plsc

_B = 16
_SEG = 2048
_D = 1024
_IMAX = 64
_HALF = _D // 2
_LANES = 16
_GROUPS = _HALF // _LANES

_RSC = 256
_CH = 64
_NCH = _RSC // _CH

_TC_BLOCKS = ((256, 1), (512, 1), (1024, 1))


def _sc_prefix_sums(hidden_states, instruction_lens):
    mesh = plsc.VectorSubcoreMesh(core_axis_name="c", subcore_axis_name="s")

    @functools.partial(
        pl.kernel,
        out_type=jax.ShapeDtypeStruct((_B, _D), jnp.float32),
        mesh=mesh,
        compiler_params=pltpu.CompilerParams(needs_layout_passes=False),
        scratch_types=[
            pltpu.VMEM((_CH, _HALF), jnp.float32),
            pltpu.VMEM((_CH, _HALF), jnp.float32),
            pltpu.VMEM((_HALF,), jnp.float32),
            pltpu.VMEM((_B,), jnp.int32),
            pltpu.SemaphoreType.DMA,
            pltpu.SemaphoreType.DMA,
        ],
    )
    def k(hs_hbm, ilen_hbm, out_hbm, buf0, buf1, acc, ilen_vmem, sem0, sem1):
        wid = lax.axis_index("s") * 2 + lax.axis_index("c")
        seg = wid // 2
        col0 = (wid % 2) * _HALF
        row0 = seg * _SEG
        bufs = (buf0, buf1)
        sems = (sem0, sem1)

        def start(c):
            return pltpu.async_copy(
                hs_hbm.at[pl.ds(row0 + c * _CH, _CH), pl.ds(col0, _HALF)],
                bufs[c % 2],
                sems[c % 2],
            )

        pltpu.sync_copy(ilen_hbm, ilen_vmem)
        handles = [start(0), start(1)]
        nvec = plsc.load_gather(ilen_vmem, [jnp.full((_LANES,), seg, jnp.int32)])
        zero = jnp.zeros((_LANES,), jnp.float32)

        for c in range(_NCH):
            buf = bufs[c % 2]
            handles[c % 2].wait()
            if c == 0:
                @pl.loop(0, _GROUPS)
                def _(g):
                    sl = pl.ds(g * _LANES, _LANES)
                    p = [zero, zero, zero, zero]
                    for r in range(_CH):
                        x = jnp.where(
                            lax.broadcast(r, (_LANES,)) >= nvec, buf[r, sl], zero
                        )
                        p[r % 4] = p[r % 4] + x
                    acc[sl] = (p[0] + p[1]) + (p[2] + p[3])
            else:
                @pl.loop(0, _GROUPS)
                def _(g):
                    sl = pl.ds(g * _LANES, _LANES)
                    p = [zero, zero, zero, zero]
                    for r in range(_CH):
                        p[r % 4] = p[r % 4] + buf[r, sl]
                    acc[sl] = acc[sl] + ((p[0] + p[1]) + (p[2] + p[3]))
            if c + 2 < _NCH:
                handles[c % 2] = start(c + 2)
        pltpu.sync_copy(acc, out_hbm.at[seg, pl.ds(col0, _HALF)])

    return k(hidden_states, instruction_lens)


def _tc_dense_body(*refs):
    hs_refs, out_ref = refs[:-1], refs[-1]
    acc = jnp.sum(hs_refs[0][...], axis=0, keepdims=True)
    for j in range(1, len(hs_refs)):
        acc = acc + jnp.sum(hs_refs[j][...], axis=0, keepdims=True)
    out_ref[0] = acc


def _tc_combine_body(plen_ref, ilen_ref, dense_ref, sc_ref, out_ref):
    for b in range(_B):
        denom = (plen_ref[b] - ilen_ref[b]).astype(jnp.float32)
        mean = (dense_ref[b, 0] + sc_ref[b]) / denom
        norm = jnp.maximum(jnp.sqrt(jnp.sum(mean * mean)), 1e-12)
        out_ref[b] = mean / norm


def kernel(hidden_states, prompt_lens, instruction_lens):
    sc_sums = _sc_prefix_sums(hidden_states, instruction_lens)
    dense_sums = pl.pallas_call(
        _tc_dense_body,
        grid=(_B,),
        in_specs=[
            pl.BlockSpec(
                (rows, _D),
                functools.partial(
                    lambda o, b: (b * (_SEG // o[0]) + o[1], 0), (rows, off)
                ),
            )
            for rows, off in _TC_BLOCKS
        ],
        out_specs=pl.BlockSpec((1, 1, _D), lambda b: (b, 0, 0)),
        out_shape=jax.ShapeDtypeStruct((_B, 1, _D), jnp.float32),
        compiler_params=pltpu.CompilerParams(dimension_semantics=("parallel",)),
    )(*([hidden_states] * len(_TC_BLOCKS)))
    return pl.pallas_call(
        _tc_combine_body,
        in_specs=[
            pl.BlockSpec(memory_space=pltpu.SMEM),
            pl.BlockSpec(memory_space=pltpu.SMEM),
            pl.BlockSpec((_B, 1, _D), lambda: (0, 0, 0)),
            pl.BlockSpec((_B, _D), lambda: (0, 0)),
        ],
        out_specs=pl.BlockSpec((_B, _D), lambda: (0, 0)),
        out_shape=jax.ShapeDtypeStruct((_B, _D), jnp.float32),
    )(prompt_lens, instruction_lens, dense_sums, sc_sums)
